# Initial kernel scaffold; baseline (speedup 1.0000x reference)
import jax
import jax.numpy as jnp
from jax.experimental import pallas as pl


def kernel(input_ids, tok_embeddings):
    raise NotImplementedError("write your pallas kernel here")


# speedup vs baseline: 1.0291x; 1.0291x over previous
import functools

import jax
import jax.numpy as jnp
from jax import lax
from jax.experimental import pallas as pl
from jax.experimental.pallas import tpu as pltpu
from jax.experimental.pallas import tpu_sc as plsc

_VOCAB = 50368
_HIDDEN = 768
_EPS = 1e-5
_LANES = 16
_NV = _HIDDEN // _LANES

_NC, _NS = 2, 16
_NW = _NC * _NS
_TOKENS = 4 * 4096
_TW = _TOKENS // _NW
_R = 128
_NCHUNK = _TW // _R


_GATHER_DNUMS = lax.GatherDimensionNumbers(
    offset_dims=(), collapsed_slice_dims=(0,), start_index_map=(0,)
)


def _permute16(v, idx):
    return lax.gather(
        v,
        idx[:, None],
        _GATHER_DNUMS,
        slice_sizes=(1,),
        mode=lax.GatherScatterMode.PROMISE_IN_BOUNDS,
    )


def _allreduce_sum16(v):
    idx = lax.iota(jnp.int32, 16)
    for off in (8, 4, 2, 1):
        v = v + _permute16(v, idx ^ off)
    return v


def _rsqrt16(x):
    i = lax.bitcast_convert_type(x, jnp.int32)
    i = jnp.int32(0x5F3759DF) - lax.shift_right_logical(i, 1)
    y = lax.bitcast_convert_type(i, jnp.float32)
    for _ in range(3):
        y = y * (1.5 - 0.5 * x * y * y)
    return y


def _body(ids_hbm, table_hbm, out_hbm, idx_v, rows_v, sem):
    wid = lax.axis_index("s") * _NC + lax.axis_index("c")
    base = wid * _TW
    pltpu.sync_copy(ids_hbm.at[pl.ds(base, _TW)], idx_v)

    for c in range(_NCHUNK):
        pltpu.async_copy(
            table_hbm.at[idx_v.at[pl.ds(c * _R, _R)]], rows_v, sem
        ).wait()

        def row_body(r, carry):
            acc = jnp.zeros((_LANES,), jnp.float32)
            acc2 = jnp.zeros((_LANES,), jnp.float32)
            for j in range(_NV):
                v = rows_v[r, pl.ds(j * _LANES, _LANES)]
                acc = acc + v
                acc2 = acc2 + v * v
            mean_v = _allreduce_sum16(acc) * (1.0 / _HIDDEN)
            var_v = _allreduce_sum16(acc2) * (1.0 / _HIDDEN) - mean_v * mean_v
            rinv_v = _rsqrt16(var_v + _EPS)
            for j in range(_NV):
                v = rows_v[r, pl.ds(j * _LANES, _LANES)]
                rows_v[r, pl.ds(j * _LANES, _LANES)] = (v - mean_v) * rinv_v
            return carry

        lax.fori_loop(0, _R, row_body, 0)
        pltpu.sync_copy(rows_v, out_hbm.at[pl.ds(base + c * _R, _R)])


_mesh = plsc.VectorSubcoreMesh(
    core_axis_name="c", subcore_axis_name="s", num_cores=_NC, num_subcores=_NS
)

_embed_ln = functools.partial(
    pl.kernel,
    out_type=jax.ShapeDtypeStruct((_TOKENS, _HIDDEN), jnp.float32),
    mesh=_mesh,
    scratch_types=[
        pltpu.VMEM((_TW,), jnp.int32),
        pltpu.VMEM((_R, _HIDDEN), jnp.float32),
        pltpu.SemaphoreType.DMA,
    ],
)(_body)


@jax.jit
def kernel(input_ids, tok_embeddings):
    b, s = input_ids.shape
    ids = input_ids.reshape(-1).astype(jnp.int32)
    out = _embed_ln(ids, tok_embeddings)
    return out.reshape(b, s, _HIDDEN)


# speedup vs baseline: 1.2054x; 1.1713x over previous
import functools

import jax
import jax.numpy as jnp
from jax import lax
from jax.experimental import pallas as pl
from jax.experimental.pallas import tpu as pltpu
from jax.experimental.pallas import tpu_sc as plsc

_HIDDEN = 768
_EPS = 1e-5
_LANES = 16
_NV = _HIDDEN // _LANES

_NC, _NS = 2, 16
_NW = _NC * _NS
_TOKENS = 4 * 4096
_TW = _TOKENS // _NW
_R = 64
_NCHUNK = _TW // _R

_GATHER_DNUMS = lax.GatherDimensionNumbers(
    offset_dims=(), collapsed_slice_dims=(0,), start_index_map=(0,)
)


def _permute16(v, idx):
    return lax.gather(
        v,
        idx[:, None],
        _GATHER_DNUMS,
        slice_sizes=(1,),
        mode=lax.GatherScatterMode.PROMISE_IN_BOUNDS,
    )


def _allreduce_sum16(v):
    idx = lax.iota(jnp.int32, 16)
    for off in (8, 4, 2, 1):
        v = v + _permute16(v, idx ^ off)
    return v


def _rsqrt16(x):
    i = lax.bitcast_convert_type(x, jnp.int32)
    i = jnp.int32(0x5F3759DF) - lax.shift_right_logical(i, 1)
    y = lax.bitcast_convert_type(i, jnp.float32)
    for _ in range(3):
        y = y * (1.5 - 0.5 * x * y * y)
    return y


def _layernorm_chunk(rows_v):
    def row_body(r, carry):
        acc = jnp.zeros((_LANES,), jnp.float32)
        acc2 = jnp.zeros((_LANES,), jnp.float32)
        for j in range(_NV):
            v = rows_v[r, pl.ds(j * _LANES, _LANES)]
            acc = acc + v
            acc2 = acc2 + v * v
        mean_v = _allreduce_sum16(acc) * (1.0 / _HIDDEN)
        var_v = _allreduce_sum16(acc2) * (1.0 / _HIDDEN) - mean_v * mean_v
        rinv_v = _rsqrt16(var_v + _EPS)
        for j in range(_NV):
            v = rows_v[r, pl.ds(j * _LANES, _LANES)]
            rows_v[r, pl.ds(j * _LANES, _LANES)] = (v - mean_v) * rinv_v
        return carry

    lax.fori_loop(0, _R, row_body, 0)


def _body(ids_hbm, table_hbm, out_hbm, idx_v, rows0, rows1, g0, g1, o0, o1):
    wid = lax.axis_index("s") * _NC + lax.axis_index("c")
    base = wid * _TW
    pltpu.sync_copy(ids_hbm.at[pl.ds(base, _TW)], idx_v)

    bufs = (rows0, rows1)
    gsems = (g0, g1)
    osems = (o0, o1)

    def gather(c, buf, sem):
        return pltpu.async_copy(
            table_hbm.at[idx_v.at[pl.ds(c * _R, _R)]], buf, sem
        )

    def writeback(c, buf, sem):
        return pltpu.async_copy(buf, out_hbm.at[pl.ds(base + c * _R, _R)], sem)

    pending_out = [None, None]
    gather(0, bufs[0], gsems[0]).wait()
    for c in range(_NCHUNK):
        cur, nxt = c % 2, (c + 1) % 2
        if c + 1 < _NCHUNK:
            if pending_out[nxt] is not None:
                pending_out[nxt].wait()
                pending_out[nxt] = None
            g = gather(c + 1, bufs[nxt], gsems[nxt])
        _layernorm_chunk(bufs[cur])
        pending_out[cur] = writeback(c, bufs[cur], osems[cur])
        if c + 1 < _NCHUNK:
            g.wait()
    for p in pending_out:
        if p is not None:
            p.wait()


_mesh = plsc.VectorSubcoreMesh(
    core_axis_name="c", subcore_axis_name="s", num_cores=_NC, num_subcores=_NS
)

_embed_ln = functools.partial(
    pl.kernel,
    out_type=jax.ShapeDtypeStruct((_TOKENS, _HIDDEN), jnp.float32),
    mesh=_mesh,
    scratch_types=[
        pltpu.VMEM((_TW,), jnp.int32),
        pltpu.VMEM((_R, _HIDDEN), jnp.float32),
        pltpu.VMEM((_R, _HIDDEN), jnp.float32),
        pltpu.SemaphoreType.DMA,
        pltpu.SemaphoreType.DMA,
        pltpu.SemaphoreType.DMA,
        pltpu.SemaphoreType.DMA,
    ],
)(_body)


@jax.jit
def kernel(input_ids, tok_embeddings):
    b, s = input_ids.shape
    ids = input_ids.reshape(-1).astype(jnp.int32)
    out = _embed_ln(ids, tok_embeddings)
    return out.reshape(b, s, _HIDDEN)


# speedup vs baseline: 1.2984x; 1.0771x over previous
import functools

import jax
import jax.numpy as jnp
from jax import lax
from jax.experimental import pallas as pl
from jax.experimental.pallas import tpu as pltpu
from jax.experimental.pallas import tpu_sc as plsc

_HIDDEN = 768
_EPS = 1e-5
_LANES = 16
_NV = _HIDDEN // _LANES

_NC, _NS = 2, 16
_NW = _NC * _NS
_TOKENS = 4 * 4096
_TW = _TOKENS // _NW
_R = 64
_NCHUNK = _TW // _R

_GATHER_DNUMS = lax.GatherDimensionNumbers(
    offset_dims=(), collapsed_slice_dims=(0,), start_index_map=(0,)
)


def _permute16(v, idx):
    return lax.gather(
        v,
        idx[:, None],
        _GATHER_DNUMS,
        slice_sizes=(1,),
        mode=lax.GatherScatterMode.PROMISE_IN_BOUNDS,
    )


def _allreduce_sum16(v):
    idx = lax.iota(jnp.int32, 16)
    for off in (8, 4, 2, 1):
        v = v + _permute16(v, idx ^ off)
    return v


def _rsqrt16(x):
    i = lax.bitcast_convert_type(x, jnp.int32)
    i = jnp.int32(0x5F3759DF) - lax.shift_right_logical(i, 1)
    y = lax.bitcast_convert_type(i, jnp.float32)
    for _ in range(3):
        y = y * (1.5 - 0.5 * x * y * y)
    return y


def _layernorm_chunk(rows_v):
    @plsc.parallel_loop(0, _R, unroll=1)
    def row_body(r):
        acc = jnp.zeros((_LANES,), jnp.float32)
        acc2 = jnp.zeros((_LANES,), jnp.float32)
        for j in range(_NV):
            v = rows_v[r, pl.ds(j * _LANES, _LANES)]
            acc = acc + v
            acc2 = acc2 + v * v
        mean_v = _allreduce_sum16(acc) * (1.0 / _HIDDEN)
        var_v = _allreduce_sum16(acc2) * (1.0 / _HIDDEN) - mean_v * mean_v
        rinv_v = _rsqrt16(var_v + _EPS)
        for j in range(_NV):
            v = rows_v[r, pl.ds(j * _LANES, _LANES)]
            rows_v[r, pl.ds(j * _LANES, _LANES)] = (v - mean_v) * rinv_v


def _body(ids_hbm, table_hbm, out_hbm, idx_v, rows0, rows1, g0, g1, o0, o1):
    wid = lax.axis_index("s") * _NC + lax.axis_index("c")
    base = wid * _TW
    pltpu.sync_copy(ids_hbm.at[pl.ds(base, _TW)], idx_v)

    bufs = (rows0, rows1)
    gsems = (g0, g1)
    osems = (o0, o1)

    def gather(c, buf, sem):
        return pltpu.async_copy(
            table_hbm.at[idx_v.at[pl.ds(c * _R, _R)]], buf, sem
        )

    def writeback(c, buf, sem):
        return pltpu.async_copy(buf, out_hbm.at[pl.ds(base + c * _R, _R)], sem)

    pending_out = [None, None]
    gather(0, bufs[0], gsems[0]).wait()
    for c in range(_NCHUNK):
        cur, nxt = c % 2, (c + 1) % 2
        if c + 1 < _NCHUNK:
            if pending_out[nxt] is not None:
                pending_out[nxt].wait()
                pending_out[nxt] = None
            g = gather(c + 1, bufs[nxt], gsems[nxt])
        _layernorm_chunk(bufs[cur])
        pending_out[cur] = writeback(c, bufs[cur], osems[cur])
        if c + 1 < _NCHUNK:
            g.wait()
    for p in pending_out:
        if p is not None:
            p.wait()


_mesh = plsc.VectorSubcoreMesh(
    core_axis_name="c", subcore_axis_name="s", num_cores=_NC, num_subcores=_NS
)

_embed_ln = functools.partial(
    pl.kernel,
    out_type=jax.ShapeDtypeStruct((_TOKENS, _HIDDEN), jnp.float32),
    mesh=_mesh,
    scratch_types=[
        pltpu.VMEM((_TW,), jnp.int32),
        pltpu.VMEM((_R, _HIDDEN), jnp.float32),
        pltpu.VMEM((_R, _HIDDEN), jnp.float32),
        pltpu.SemaphoreType.DMA,
        pltpu.SemaphoreType.DMA,
        pltpu.SemaphoreType.DMA,
        pltpu.SemaphoreType.DMA,
    ],
)(_body)


@jax.jit
def kernel(input_ids, tok_embeddings):
    b, s = input_ids.shape
    ids = input_ids.reshape(-1).astype(jnp.int32)
    out = _embed_ln(ids, tok_embeddings)
    return out.reshape(b, s, _HIDDEN)


# speedup vs baseline: 1.5604x; 1.2018x over previous
import functools

import jax
import jax.numpy as jnp
from jax import lax
from jax.experimental import pallas as pl
from jax.experimental.pallas import tpu as pltpu
from jax.experimental.pallas import tpu_sc as plsc

_HIDDEN = 768
_EPS = 1e-5
_LANES = 16
_NV = _HIDDEN // _LANES

_NC, _NS = 2, 16
_NW = _NC * _NS
_TOKENS = 4 * 4096
_TW = _TOKENS // _NW
_R = 64
_NCHUNK = _TW // _R

_GATHER_DNUMS = lax.GatherDimensionNumbers(
    offset_dims=(), collapsed_slice_dims=(0,), start_index_map=(0,)
)


def _permute16(v, idx):
    return lax.gather(
        v,
        idx[:, None],
        _GATHER_DNUMS,
        slice_sizes=(1,),
        mode=lax.GatherScatterMode.PROMISE_IN_BOUNDS,
    )


def _allreduce_sum16(v):
    idx = lax.iota(jnp.int32, 16)
    for off in (8, 4, 2, 1):
        v = v + _permute16(v, idx ^ off)
    return v


def _rsqrt16(x):
    i = lax.bitcast_convert_type(x, jnp.int32)
    i = jnp.int32(0x5F3759DF) - lax.shift_right_logical(i, 1)
    y = lax.bitcast_convert_type(i, jnp.float32)
    for _ in range(3):
        y = y * (1.5 - 0.5 * x * y * y)
    return y


def _layernorm_chunk(rows_v):
    @plsc.parallel_loop(0, _R, unroll=2)
    def row_body(r):
        acc = jnp.zeros((_LANES,), jnp.float32)
        acc2 = jnp.zeros((_LANES,), jnp.float32)
        for j in range(_NV):
            v = rows_v[r, pl.ds(j * _LANES, _LANES)]
            acc = acc + v
            acc2 = acc2 + v * v
        mean_v = _allreduce_sum16(acc) * (1.0 / _HIDDEN)
        var_v = _allreduce_sum16(acc2) * (1.0 / _HIDDEN) - mean_v * mean_v
        rinv_v = _rsqrt16(var_v + _EPS)
        for j in range(_NV):
            v = rows_v[r, pl.ds(j * _LANES, _LANES)]
            rows_v[r, pl.ds(j * _LANES, _LANES)] = (v - mean_v) * rinv_v


def _body(ids_hbm, table_hbm, out_hbm, idx_v, rows0, rows1, g0, g1, o0, o1):
    wid = lax.axis_index("s") * _NC + lax.axis_index("c")
    base = wid * _TW
    pltpu.sync_copy(ids_hbm.at[pl.ds(base, _TW)], idx_v)

    def gather(c, buf, sem):
        return pltpu.make_async_copy(
            table_hbm.at[idx_v.at[pl.ds(c * _R, _R)]], buf, sem
        )

    def writeback(c, buf, sem):
        return pltpu.make_async_copy(
            buf, out_hbm.at[pl.ds(base + c * _R, _R)], sem
        )

    nsuper = _NCHUNK // 2
    gather(0, rows0, g0).start()

    def superstep(s, carry):
        a = 2 * s
        b = a + 1
        gather(a, rows0, g0).wait()

        @pl.when(s > 0)
        def _():
            writeback(a - 1, rows1, o1).wait()

        gather(b, rows1, g1).start()
        _layernorm_chunk(rows0)
        writeback(a, rows0, o0).start()
        gather(b, rows1, g1).wait()
        _layernorm_chunk(rows1)
        writeback(b, rows1, o1).start()

        @pl.when(s < nsuper - 1)
        def _():
            writeback(a, rows0, o0).wait()
            gather(a + 2, rows0, g0).start()

        return carry

    lax.fori_loop(0, nsuper, superstep, 0)
    writeback(2 * nsuper - 2, rows0, o0).wait()
    writeback(2 * nsuper - 1, rows1, o1).wait()


_mesh = plsc.VectorSubcoreMesh(
    core_axis_name="c", subcore_axis_name="s", num_cores=_NC, num_subcores=_NS
)

_embed_ln = functools.partial(
    pl.kernel,
    out_type=jax.ShapeDtypeStruct((_TOKENS, _HIDDEN), jnp.float32),
    mesh=_mesh,
    scratch_types=[
        pltpu.VMEM((_TW,), jnp.int32),
        pltpu.VMEM((_R, _HIDDEN), jnp.float32),
        pltpu.VMEM((_R, _HIDDEN), jnp.float32),
        pltpu.SemaphoreType.DMA,
        pltpu.SemaphoreType.DMA,
        pltpu.SemaphoreType.DMA,
        pltpu.SemaphoreType.DMA,
    ],
)(_body)


@jax.jit
def kernel(input_ids, tok_embeddings):
    b, s = input_ids.shape
    ids = input_ids.reshape(-1).astype(jnp.int32)
    out = _embed_ln(ids, tok_embeddings)
    return out.reshape(b, s, _HIDDEN)


# speedup vs baseline: 1.5671x; 1.0043x over previous
import functools

import jax
import jax.numpy as jnp
from jax import lax
from jax.experimental import pallas as pl
from jax.experimental.pallas import tpu as pltpu
from jax.experimental.pallas import tpu_sc as plsc

_HIDDEN = 768
_EPS = 1e-5
_LANES = 16
_NV = _HIDDEN // _LANES

_NC, _NS = 2, 16
_NW = _NC * _NS
_TOKENS = 4 * 4096
_TW = _TOKENS // _NW
_R = 64
_NCHUNK = _TW // _R

_GATHER_DNUMS = lax.GatherDimensionNumbers(
    offset_dims=(), collapsed_slice_dims=(0,), start_index_map=(0,)
)


def _permute16(v, idx):
    return lax.gather(
        v,
        idx[:, None],
        _GATHER_DNUMS,
        slice_sizes=(1,),
        mode=lax.GatherScatterMode.PROMISE_IN_BOUNDS,
    )


def _allreduce_sum16(v):
    idx = lax.iota(jnp.int32, 16)
    for off in (8, 4, 2, 1):
        v = v + _permute16(v, idx ^ off)
    return v


def _rsqrt16(x):
    i = lax.bitcast_convert_type(x, jnp.int32)
    i = jnp.int32(0x5F3759DF) - lax.shift_right_logical(i, 1)
    y = lax.bitcast_convert_type(i, jnp.float32)
    for _ in range(3):
        y = y * (1.5 - 0.5 * x * y * y)
    return y


def _layernorm_chunk(rows_v):
    @plsc.parallel_loop(0, _R, unroll=4)
    def row_body(r):
        acc = jnp.zeros((_LANES,), jnp.float32)
        acc2 = jnp.zeros((_LANES,), jnp.float32)
        for j in range(_NV):
            v = rows_v[r, pl.ds(j * _LANES, _LANES)]
            acc = acc + v
            acc2 = acc2 + v * v
        mean_v = _allreduce_sum16(acc) * (1.0 / _HIDDEN)
        var_v = _allreduce_sum16(acc2) * (1.0 / _HIDDEN) - mean_v * mean_v
        rinv_v = _rsqrt16(var_v + _EPS)
        for j in range(_NV):
            v = rows_v[r, pl.ds(j * _LANES, _LANES)]
            rows_v[r, pl.ds(j * _LANES, _LANES)] = (v - mean_v) * rinv_v


def _body(ids_hbm, table_hbm, out_hbm, idx_v, rows0, rows1, g0, g1, o0, o1):
    wid = lax.axis_index("s") * _NC + lax.axis_index("c")
    base = wid * _TW
    pltpu.sync_copy(ids_hbm.at[pl.ds(base, _TW)], idx_v)

    def gather(c, buf, sem):
        return pltpu.make_async_copy(
            table_hbm.at[idx_v.at[pl.ds(c * _R, _R)]], buf, sem
        )

    def writeback(c, buf, sem):
        return pltpu.make_async_copy(
            buf, out_hbm.at[pl.ds(base + c * _R, _R)], sem
        )

    nsuper = _NCHUNK // 2
    gather(0, rows0, g0).start()

    def superstep(s, carry):
        a = 2 * s
        b = a + 1
        gather(a, rows0, g0).wait()

        @pl.when(s > 0)
        def _():
            writeback(a - 1, rows1, o1).wait()

        gather(b, rows1, g1).start()
        _layernorm_chunk(rows0)
        writeback(a, rows0, o0).start()
        gather(b, rows1, g1).wait()
        _layernorm_chunk(rows1)
        writeback(b, rows1, o1).start()

        @pl.when(s < nsuper - 1)
        def _():
            writeback(a, rows0, o0).wait()
            gather(a + 2, rows0, g0).start()

        return carry

    lax.fori_loop(0, nsuper, superstep, 0)
    writeback(2 * nsuper - 2, rows0, o0).wait()
    writeback(2 * nsuper - 1, rows1, o1).wait()


_mesh = plsc.VectorSubcoreMesh(
    core_axis_name="c", subcore_axis_name="s", num_cores=_NC, num_subcores=_NS
)

_embed_ln = functools.partial(
    pl.kernel,
    out_type=jax.ShapeDtypeStruct((_TOKENS, _HIDDEN), jnp.float32),
    mesh=_mesh,
    scratch_types=[
        pltpu.VMEM((_TW,), jnp.int32),
        pltpu.VMEM((_R, _HIDDEN), jnp.float32),
        pltpu.VMEM((_R, _HIDDEN), jnp.float32),
        pltpu.SemaphoreType.DMA,
        pltpu.SemaphoreType.DMA,
        pltpu.SemaphoreType.DMA,
        pltpu.SemaphoreType.DMA,
    ],
)(_body)


@jax.jit
def kernel(input_ids, tok_embeddings):
    b, s = input_ids.shape
    ids = input_ids.reshape(-1).astype(jnp.int32)
    out = _embed_ln(ids, tok_embeddings)
    return out.reshape(b, s, _HIDDEN)
